# baseline (speedup 1.0000x reference)
import jax
import jax.numpy as jnp
from jax import lax
from jax.experimental import pallas as pl
from jax.experimental.pallas import tpu as pltpu
from jax.experimental.pallas import tpu_sc as plsc

_BLK = 1024
_NC = 2
_NS = 16
_NW = _NC * _NS


def kernel(position_idx, table):
    batch, seq = position_idx.shape
    n = batch * seq
    dim = table.shape[1]
    idx = position_idx.reshape(1, n)

    per_w = n // _NW
    nblk = per_w // _BLK

    mesh = plsc.VectorSubcoreMesh(core_axis_name="core",
                                  subcore_axis_name="subcore")

    @jax.jit
    def run(table_arr, idx_arr):
        @pl.kernel(out_type=jax.ShapeDtypeStruct((n, dim),
                                                 table_arr.dtype),
                   mesh=mesh,
                   scratch_types=[
                       pltpu.VMEM((2, 1, _BLK), jnp.int32),
                       pltpu.VMEM((2, _BLK, dim), jnp.float32),
                       pltpu.SemaphoreType.DMA,
                       pltpu.SemaphoreType.DMA,
                       pltpu.SemaphoreType.DMA,
                   ],
                   compiler_params=pltpu.CompilerParams(
                       use_tc_tiling_on_sc=False))
        def gather_kernel(table_hbm, idx_hbm, out_hbm, idx_v, out_v,
                          sem_g, sem_o0, sem_o1):
            wid = lax.axis_index("subcore") * _NC + lax.axis_index("core")
            base = wid * per_w
            sems = (sem_o0, sem_o1)

            @pl.loop(0, nblk, step=2)
            def _(i):
                for r in range(2):
                    b = i + r
                    off = base + b * _BLK

                    @pl.when(b >= 2)
                    def _():
                        pltpu.make_async_copy(
                            out_v.at[r],
                            out_hbm.at[pl.ds(off - 2 * _BLK, _BLK)],
                            sems[r],
                        ).wait()

                    pltpu.sync_copy(idx_hbm.at[0, pl.ds(off, _BLK)],
                                    idx_v.at[r, 0])

                    pltpu.async_copy(
                        table_hbm.at[idx_v.at[r, 0]],
                        out_v.at[r],
                        sem_g,
                    ).wait()

                    pltpu.async_copy(out_v.at[r],
                                     out_hbm.at[pl.ds(off, _BLK)],
                                     sems[r])

            for r in range(2):
                last_off = base + (nblk - 2 + r) * _BLK
                pltpu.make_async_copy(
                    out_v.at[r],
                    out_hbm.at[pl.ds(last_off, _BLK)],
                    sems[r],
                ).wait()

        return gather_kernel(table_arr, idx_arr)

    return run(table, idx).reshape(batch, seq, dim)


# speedup vs baseline: 2.7419x; 2.7419x over previous
import jax
import jax.numpy as jnp
from jax import lax
from jax.experimental import pallas as pl
from jax.experimental.pallas import tpu as pltpu
from jax.experimental.pallas import tpu_sc as plsc

_BLK = 1024
_BT = 256
_NC = 2
_NS = 16
_NW = _NC * _NS


def kernel(position_idx, table):
    batch, seq = position_idx.shape
    n = batch * seq
    dim = table.shape[1]
    idx = position_idx.reshape(1, n)

    per_w = n // _NW
    nblk = per_w // _BLK

    mesh = plsc.VectorSubcoreMesh(core_axis_name="core",
                                  subcore_axis_name="subcore")

    @jax.jit
    def run(table_arr, idx_arr):
        @pl.kernel(out_type=jax.ShapeDtypeStruct((n, dim),
                                                 table_arr.dtype),
                   mesh=mesh,
                   scratch_types=[
                       pltpu.VMEM((2, 1, _BLK), jnp.int32),
                       pltpu.VMEM((2, _BLK, dim), jnp.float32),
                       pltpu.SemaphoreType.DMA,
                       pltpu.SemaphoreType.DMA,
                       pltpu.SemaphoreType.DMA,
                   ],
                   compiler_params=pltpu.CompilerParams(
                       use_tc_tiling_on_sc=False))
        def gather_kernel(table_hbm, idx_hbm, out_hbm, idx_v, out_v,
                          sem_g, sem_o0, sem_o1):
            wid = lax.axis_index("subcore") * _NC + lax.axis_index("core")
            base = wid * per_w
            sems = (sem_o0, sem_o1)

            @pl.loop(0, nblk, step=2)
            def _(i):
                for r in range(2):
                    b = i + r
                    off = base + b * _BLK

                    @pl.when(b >= 2)
                    def _():
                        pltpu.make_async_copy(
                            out_v.at[r],
                            out_hbm.at[pl.ds(off - 2 * _BLK, _BLK)],
                            sems[r],
                        ).wait()

                    pltpu.sync_copy(idx_hbm.at[0, pl.ds(off, _BLK)],
                                    idx_v.at[r, 0])

                    pltpu.async_copy(
                        table_hbm.at[idx_v.at[r, 0]],
                        out_v.at[r],
                        sem_g,
                    ).wait()

                    pltpu.async_copy(out_v.at[r],
                                     out_hbm.at[pl.ds(off, _BLK)],
                                     sems[r])

            for r in range(2):
                last_off = base + (nblk - 2 + r) * _BLK
                pltpu.make_async_copy(
                    out_v.at[r],
                    out_hbm.at[pl.ds(last_off, _BLK)],
                    sems[r],
                ).wait()

        return gather_kernel(table_arr, idx_arr)

    flat = run(table, idx)
    pack = 128 // dim
    njt = seq * dim // 128
    g = flat.reshape(n // pack, 128)

    def _transpose_body(g_ref, o_ref):
        x3 = g_ref[...].reshape(_BT, njt, 128)
        for j in range(njt):
            o_ref[j] = x3[:, j, :].T

    out3 = pl.pallas_call(
        _transpose_body,
        grid=(batch // _BT,),
        in_specs=[pl.BlockSpec((_BT * njt, 128), lambda i: (i, 0))],
        out_specs=pl.BlockSpec((njt, 128, _BT), lambda i: (0, 0, i)),
        out_shape=jax.ShapeDtypeStruct((njt, 128, batch), jnp.float32),
        compiler_params=pltpu.CompilerParams(
            dimension_semantics=("arbitrary",)),
    )(g)

    return out3.reshape(seq, dim, batch).transpose(2, 0, 1)
